# baseline (speedup 1.0000x reference)
import jax
import jax.numpy as jnp
from jax import lax
from jax.experimental import pallas as pl
from jax.experimental.pallas import tpu as pltpu, tpu_sc as plsc

D = 64
S = 200
B = 4096
NC = 2
NS = 16
NW = NC * NS
CBT = 2
CB = CBT * 128
NCH = B // CB
UNITS = S * NCH // NW
DH, DL = D // 8, 8
BT = B // 128


def _body(idsT_hbm, pos_splat_hbm, tok_hbm, out_hbm,
          idxv, gbuf, psv, stage, gsem, wsem):
    wid = lax.axis_index("s") * NC + lax.axis_index("c")
    base = wid * UNITS
    iota = lax.iota(jnp.int32, 16)

    def unit_sc(j):
        u = base + j
        return u // NCH, lax.rem(u, NCH)

    def fire(j):
        s, c = unit_sc(j)
        n = lax.rem(j, 2)
        pltpu.sync_copy(idsT_hbm.at[s, pl.ds(c * CB, 128)], idxv.at[n, 0])
        pltpu.sync_copy(idsT_hbm.at[s, pl.ds(c * CB + 128, 128)], idxv.at[n, 1])
        pltpu.async_copy(tok_hbm.at[idxv.at[n, 0]], gbuf.at[n, 0], gsem.at[n])
        pltpu.async_copy(tok_hbm.at[idxv.at[n, 1]], gbuf.at[n, 1], gsem.at[n])

    def transpose(j):
        s, c = unit_sc(j)
        n = lax.rem(j, 2)
        pltpu.sync_copy(pos_splat_hbm.at[s], psv)
        pltpu.make_async_copy(tok_hbm.at[idxv.at[n, 0]], gbuf.at[n, 0],
                              gsem.at[n]).wait()
        pltpu.make_async_copy(tok_hbm.at[idxv.at[n, 1]], gbuf.at[n, 1],
                              gsem.at[n]).wait()
        z16 = jnp.zeros((16,), jnp.int32)
        nv = z16 + n

        def gloop(g, _):
            k = g // 8
            kv = z16 + k
            rowv = lax.rem(g, 8) * 16 + iota

            def dloop(d, _):
                dv = z16 + d
                val = plsc.load_gather(gbuf, [nv, kv, rowv, dv])
                val = val + psv[d]
                stage[n, d // 8, k, lax.rem(d, 8),
                      pl.ds(lax.rem(g, 8) * 16, 16)] = val
                return 0

            lax.fori_loop(0, D, dloop, 0)
            return 0

        lax.fori_loop(0, 16, gloop, 0)

    def outfire(j):
        s, c = unit_sc(j)
        n = lax.rem(j, 2)
        for dh in range(DH):
            pltpu.async_copy(stage.at[n, dh],
                             out_hbm.at[s, dh, pl.ds(CBT * c, CBT)],
                             wsem.at[n])

    def outwait(j):
        s, c = unit_sc(j)
        n = lax.rem(j, 2)
        for dh in range(DH):
            pltpu.make_async_copy(stage.at[n, dh],
                                  out_hbm.at[s, dh, pl.ds(CBT * c, CBT)],
                                  wsem.at[n]).wait()

    def step(j, carry):
        pl.when(j + 1 < UNITS)(lambda: fire(j + 1))
        pl.when(j >= 2)(lambda: outwait(j - 2))
        transpose(j)
        outfire(j)
        return carry

    fire(0)
    lax.fori_loop(0, UNITS, step, 0)
    outwait(UNITS - 2)
    outwait(UNITS - 1)


def kernel(input_ids, token_embedding, position_embedding):
    ids_t = input_ids.astype(jnp.int32).T
    pos_splat = jnp.broadcast_to(position_embedding[:S, :, None], (S, D, 16))
    mesh = plsc.VectorSubcoreMesh(core_axis_name="c", subcore_axis_name="s")
    h = pl.kernel(
        _body,
        out_type=jax.ShapeDtypeStruct((S, DH, BT, DL, 128), jnp.float32),
        mesh=mesh,
        scratch_types=[
            pltpu.VMEM((2, 2, 128), jnp.int32),
            pltpu.VMEM((2, 2, 128, D), jnp.float32),
            pltpu.VMEM((D, 16), jnp.float32),
            pltpu.VMEM((2, DH, CBT, DL, 128), jnp.float32),
            pltpu.SemaphoreType.DMA((2,)),
            pltpu.SemaphoreType.DMA((2,)),
        ],
        compiler_params=pltpu.CompilerParams(
            use_tc_tiling_on_sc=False, needs_layout_passes=False),
    )(ids_t, pos_splat, token_embedding)
    return h.transpose(2, 4, 0, 1, 3).reshape(B, S, D)


# speedup vs baseline: 1.0705x; 1.0705x over previous
import jax
import jax.numpy as jnp
from jax import lax
from jax.experimental import pallas as pl
from jax.experimental.pallas import tpu as pltpu, tpu_sc as plsc

D = 64
S = 200
B = 4096
NC = 2
NS = 16
NW = NC * NS
CBT = 2
CB = CBT * 128
NCH = B // CB
UNITS = S * NCH // NW
DH, DL = D // 8, 8
BT = B // 128


def _body(idsT_hbm, pos_splat_hbm, tok_hbm, out_hbm,
          idxv, gbuf, psv, stage, gsem, wsem):
    wid = lax.axis_index("s") * NC + lax.axis_index("c")
    base = wid * UNITS
    iota = lax.iota(jnp.int32, 16)

    def unit_sc(j):
        u = base + j
        return u // NCH, lax.rem(u, NCH)

    def fire(j):
        s, c = unit_sc(j)
        n = lax.rem(j, 2)
        pltpu.sync_copy(idsT_hbm.at[s, pl.ds(c * CB, 128)], idxv.at[n, 0])
        pltpu.sync_copy(idsT_hbm.at[s, pl.ds(c * CB + 128, 128)], idxv.at[n, 1])
        pltpu.async_copy(tok_hbm.at[idxv.at[n, 0]], gbuf.at[n, 0], gsem.at[n])
        pltpu.async_copy(tok_hbm.at[idxv.at[n, 1]], gbuf.at[n, 1], gsem.at[n])

    def transpose(j, s0):
        s, c = unit_sc(j)
        n = lax.rem(j, 2)
        pltpu.make_async_copy(tok_hbm.at[idxv.at[n, 0]], gbuf.at[n, 0],
                              gsem.at[n]).wait()
        pltpu.make_async_copy(tok_hbm.at[idxv.at[n, 1]], gbuf.at[n, 1],
                              gsem.at[n]).wait()
        z16 = jnp.zeros((16,), jnp.int32)
        nv = z16 + n
        sl = s - s0

        def dloop(d, _):
            dv = z16 + d
            pv = psv[sl, d]
            dh = d // 8
            dl = lax.rem(d, 8)
            for g in range(16):
                k = g // 8
                rowv = iota + (g % 8) * 16
                val = plsc.load_gather(gbuf, [nv, z16 + k, rowv, dv])
                stage[n, dh, k, dl, pl.ds((g % 8) * 16, 16)] = val + pv
            return 0

        lax.fori_loop(0, D, dloop, 0)

    def outfire(j):
        s, c = unit_sc(j)
        n = lax.rem(j, 2)
        for dh in range(DH):
            pltpu.async_copy(stage.at[n, dh],
                             out_hbm.at[s, dh, pl.ds(CBT * c, CBT)],
                             wsem.at[n])

    def outwait(j):
        s, c = unit_sc(j)
        n = lax.rem(j, 2)
        for dh in range(DH):
            pltpu.make_async_copy(stage.at[n, dh],
                                  out_hbm.at[s, dh, pl.ds(CBT * c, CBT)],
                                  wsem.at[n]).wait()

    s0 = base // NCH
    pltpu.sync_copy(pos_splat_hbm.at[pl.ds(s0, 8)], psv)

    def step(j, carry):
        pl.when(j + 1 < UNITS)(lambda: fire(j + 1))
        pl.when(j >= 2)(lambda: outwait(j - 2))
        transpose(j, s0)
        outfire(j)
        return carry

    fire(0)
    lax.fori_loop(0, UNITS, step, 0)
    outwait(UNITS - 2)
    outwait(UNITS - 1)


def kernel(input_ids, token_embedding, position_embedding):
    ids_t = input_ids.astype(jnp.int32).T
    pos_splat = jnp.concatenate(
        [jnp.broadcast_to(position_embedding[:S, :, None], (S, D, 16)),
         jnp.zeros((8, D, 16), jnp.float32)])
    mesh = plsc.VectorSubcoreMesh(core_axis_name="c", subcore_axis_name="s")
    h = pl.kernel(
        _body,
        out_type=jax.ShapeDtypeStruct((S, DH, BT, DL, 128), jnp.float32),
        mesh=mesh,
        scratch_types=[
            pltpu.VMEM((2, 2, 128), jnp.int32),
            pltpu.VMEM((2, 2, 128, D), jnp.float32),
            pltpu.VMEM((8, D, 16), jnp.float32),
            pltpu.VMEM((2, DH, CBT, DL, 128), jnp.float32),
            pltpu.SemaphoreType.DMA((2,)),
            pltpu.SemaphoreType.DMA((2,)),
        ],
        compiler_params=pltpu.CompilerParams(
            use_tc_tiling_on_sc=False, needs_layout_passes=False),
    )(ids_t, pos_splat, token_embedding)
    return h.transpose(2, 4, 0, 1, 3).reshape(B, S, D)


# speedup vs baseline: 1.6959x; 1.5842x over previous
import jax
import jax.numpy as jnp
from jax import lax
from jax.experimental import pallas as pl
from jax.experimental.pallas import tpu as pltpu, tpu_sc as plsc

D = 64
S = 200
B = 4096
NC = 2
NS = 16
NW = NC * NS
ROWS = B // NW
CH = S // 2
NBUF = 3


def _body(ids_hbm, tok_hbm, pos_hbm, out_hbm,
          idx_all, posb, buf, gsem, wsem):
    wid = lax.axis_index("s") * NC + lax.axis_index("c")
    row0 = wid * ROWS
    pltpu.sync_copy(pos_hbm.at[pl.ds(0, S)], posb)
    pltpu.sync_copy(ids_hbm.at[pl.ds(row0, ROWS)], idx_all)

    def init_and_fire(t):
        n = lax.rem(t, NBUF)

        def rloop(r2, _):
            k = r2 // CH
            r = lax.rem(r2, CH)
            for q in range(D // 16):
                buf[n, k, r, pl.ds(16 * q, 16)] = posb[r2, pl.ds(16 * q, 16)]
            return 0

        lax.fori_loop(0, S, rloop, 0)
        pltpu.async_copy(tok_hbm.at[idx_all.at[t, 0]], buf.at[n, 0],
                         gsem.at[n], add=True)
        pltpu.async_copy(tok_hbm.at[idx_all.at[t, 1]], buf.at[n, 1],
                         gsem.at[n], add=True)

    def drain(t):
        n = lax.rem(t, NBUF)
        pltpu.make_async_copy(tok_hbm.at[idx_all.at[t, 0]], buf.at[n, 0],
                              gsem.at[n]).wait()
        pltpu.make_async_copy(tok_hbm.at[idx_all.at[t, 1]], buf.at[n, 1],
                              gsem.at[n]).wait()
        pltpu.async_copy(buf.at[n, 0], out_hbm.at[row0 + t, pl.ds(0, CH)],
                         wsem.at[n])
        pltpu.async_copy(buf.at[n, 1], out_hbm.at[row0 + t, pl.ds(CH, CH)],
                         wsem.at[n])

    def flush(t):
        n = lax.rem(t, NBUF)
        pltpu.make_async_copy(buf.at[n, 0], out_hbm.at[row0 + t, pl.ds(0, CH)],
                              wsem.at[n]).wait()
        pltpu.make_async_copy(buf.at[n, 1], out_hbm.at[row0 + t, pl.ds(CH, CH)],
                              wsem.at[n]).wait()

    def step(t, carry):
        pl.when(t >= NBUF)(lambda: flush(t - NBUF))
        pl.when(t < ROWS)(lambda: init_and_fire(t))
        pl.when(jnp.logical_and(t >= 1, t < ROWS + 1))(lambda: drain(t - 1))
        return carry

    lax.fori_loop(0, ROWS + NBUF, step, 0)


def kernel(input_ids, token_embedding, position_embedding):
    ids = input_ids.astype(jnp.int32).reshape(B, 2, CH)
    mesh = plsc.VectorSubcoreMesh(core_axis_name="c", subcore_axis_name="s")
    out = pl.kernel(
        _body,
        out_type=jax.ShapeDtypeStruct((B, S, D), jnp.float32),
        mesh=mesh,
        scratch_types=[
            pltpu.VMEM((ROWS, 2, CH), jnp.int32),
            pltpu.VMEM((S, D), jnp.float32),
            pltpu.VMEM((NBUF, 2, CH, D), jnp.float32),
            pltpu.SemaphoreType.DMA((NBUF,)),
            pltpu.SemaphoreType.DMA((NBUF,)),
        ],
        compiler_params=pltpu.CompilerParams(use_tc_tiling_on_sc=False),
    )(ids, token_embedding, position_embedding)
    return out
